# baseline (speedup 1.0000x reference)
import functools

import jax
import jax.numpy as jnp
from jax import lax
from jax.experimental import pallas as pl
from jax.experimental.pallas import tpu as pltpu
from jax.experimental.pallas import tpu_sc as plsc

H = 64
BASKET = 20
NC, NS = 2, 16
NW = NC * NS
LANES = 16


def _sc_gather_kernel(batch):
    b_per_w = batch // NW
    n_chunks = 4
    rows_per_chunk = b_per_w // n_chunks
    s_chunk = rows_per_chunk * BASKET

    mesh = plsc.VectorSubcoreMesh(
        core_axis_name="c", subcore_axis_name="s",
        num_cores=NC, num_subcores=NS)

    @functools.partial(
        pl.kernel,
        out_type=(
            jax.ShapeDtypeStruct((batch, H), jnp.float32),
            jax.ShapeDtypeStruct((batch, H), jnp.float32),
        ),
        mesh=mesh,
        scratch_types=dict(
            u_idx=pltpu.VMEM((b_per_w,), jnp.int32),
            a_idx=pltpu.VMEM((b_per_w,), jnp.int32),
            s_idx=pltpu.VMEM((s_chunk,), jnp.int32),
            usr_rows=pltpu.VMEM((b_per_w, H), jnp.float32),
            a_rows=pltpu.VMEM((b_per_w, H), jnp.float32),
            s_rows=pltpu.VMEM((s_chunk, H), jnp.float32),
            lhs_buf=pltpu.VMEM((b_per_w, H), jnp.float32),
            sem_u=pltpu.SemaphoreType.DMA,
            sem_a=pltpu.SemaphoreType.DMA,
            sem_s=pltpu.SemaphoreType.DMA,
        ),
        compiler_params=pltpu.CompilerParams(use_tc_tiling_on_sc=False),
    )
    def sc_fn(u_hbm, a_hbm, s_hbm, item_hbm, usr_hbm, lhs_hbm, rhs_hbm,
              u_idx, a_idx, s_idx, usr_rows, a_rows, s_rows, lhs_buf,
              sem_u, sem_a, sem_s):
        wid = lax.axis_index("s") * NC + lax.axis_index("c")
        base = wid * b_per_w

        pltpu.sync_copy(u_hbm.at[pl.ds(base, b_per_w)], u_idx)
        pltpu.sync_copy(a_hbm.at[pl.ds(base, b_per_w)], a_idx)

        cp_u = pltpu.async_copy(usr_hbm.at[u_idx], usr_rows, sem_u)
        cp_a = pltpu.async_copy(item_hbm.at[a_idx], a_rows, sem_a)

        for c in range(n_chunks):
            pltpu.sync_copy(
                s_hbm.at[pl.ds(base * BASKET + c * s_chunk, s_chunk)], s_idx)
            pltpu.async_copy(item_hbm.at[s_idx], s_rows, sem_s).wait()
            if c == 0:
                cp_u.wait()

            def body(r, _):
                row = c * rows_per_chunk + r
                for h in range(H // LANES):
                    acc = usr_rows[row, pl.ds(h * LANES, LANES)]
                    for j in range(BASKET):
                        acc = acc + s_rows[r * BASKET + j,
                                           pl.ds(h * LANES, LANES)]
                    lhs_buf[row, pl.ds(h * LANES, LANES)] = acc
                return _

            lax.fori_loop(0, rows_per_chunk, body, None, unroll=False)

        cp_a.wait()
        pltpu.sync_copy(lhs_buf, lhs_hbm.at[pl.ds(base, b_per_w)])
        pltpu.sync_copy(a_rows, rhs_hbm.at[pl.ds(base, b_per_w)])

    return sc_fn


def _mm_body(lhs_ref, rhs_ref, out_ref):
    out_ref[...] = lax.dot_general(
        lhs_ref[...], rhs_ref[...],
        dimension_numbers=(((1,), (1,)), ((), ())),
        preferred_element_type=jnp.float32,
    )


def _tc_matmul(lhs, rhs, blk_m=512, blk_n=1024):
    batch = lhs.shape[0]
    grid = (batch // blk_m, batch // blk_n)
    return pl.pallas_call(
        _mm_body,
        out_shape=jax.ShapeDtypeStruct((batch, batch), jnp.float32),
        grid=grid,
        in_specs=[
            pl.BlockSpec((blk_m, H), lambda i, j: (i, 0)),
            pl.BlockSpec((blk_n, H), lambda i, j: (j, 0)),
        ],
        out_specs=pl.BlockSpec((blk_m, blk_n), lambda i, j: (i, j)),
    )(lhs, rhs)


@jax.jit
def kernel(U, S, A, B, item_embedding, usr_embedding):
    batch = U.shape[0]
    s_last = S[:, -1, :].reshape(-1).astype(jnp.int32)
    lhs, rhs = _sc_gather_kernel(batch)(
        U.astype(jnp.int32), A.astype(jnp.int32), s_last,
        item_embedding, usr_embedding)
    return _tc_matmul(lhs, rhs)
